# baseline (speedup 1.0000x reference)
import functools

import jax
import jax.numpy as jnp
from jax import lax
from jax.experimental import pallas as pl
from jax.experimental.pallas import tpu as pltpu
from jax.experimental.pallas import tpu_sc as plsc

B, N, C = 8, 20000, 91
L = 16
NC, NS = 2, 16
NW = NC * NS
WPI = NW // B
GPI = N // L
WG = -(-GPI // WPI)
K = 16
NCHUNK = -(-WG // K)
CHB = K * L
OFFS = (0, 16, 32, 48, 64, 75)


def _permute(g, idx):
  dn = lax.GatherDimensionNumbers(offset_dims=(), collapsed_slice_dims=(0,),
                                  start_index_map=(0,))
  return lax.gather(g, idx[:, None], dn, (1,),
                    mode=lax.GatherScatterMode.PROMISE_IN_BOUNDS)


def _sc_body(lg_hbm, pr_hbm, sc_hbm,
             lg_v, sb_v, pr_v, a_v, sem0, sem1, semw0, semw1):
  wid = lax.axis_index("s") * NC + lax.axis_index("c")
  img = wid // WPI
  q = wid % WPI
  g0 = jnp.minimum(q * WG, GPI - WG)
  n0 = g0 * L

  iota = lax.iota(jnp.int32, L)
  perms = {k: iota ^ k for k in (8, 4, 2, 1)}
  masks = {k: (iota & k) == 0 for k in (8, 4, 2, 1)}
  bitrev = (((iota & 1) << 3) | ((iota & 2) << 1)
            | ((iota & 4) >> 1) | ((iota & 8) >> 3))

  pltpu.sync_copy(pr_hbm.at[img], pr_v)
  for off in OFFS:
    p = pr_v[pl.ds(off, L)]
    a_v[pl.ds(off, L)] = 1.0 + jnp.exp(-p)
  avecs = [a_v[pl.ds(off, L)] for off in OFFS]

  def issue_in(t, half, sem):
    cg = jnp.minimum(t * K, WG - K)
    pltpu.async_copy(lg_hbm.at[img, pl.ds(n0 + cg * L, CHB), :],
                     lg_v.at[pl.ds(half * CHB, CHB), :], sem)

  issue_in(0, 0, sem0)
  issue_in(1, 1, sem1)

  def chunk(t, half, sem, semw, first):
    cg = jnp.minimum(t * K, WG - K)
    boxb = img * N + n0 + cg * L
    pltpu.make_async_copy(lg_hbm.at[img, pl.ds(0, CHB), :],
                          lg_v.at[pl.ds(half * CHB, CHB), :], sem).wait()

    @pl.when(jnp.logical_not(first))
    def _drain():
      pltpu.make_async_copy(sb_v.at[pl.ds(half * CHB, CHB)],
                            sc_hbm.at[pl.ds(0, CHB)], semw).wait()

    def grp_body(g, carry):
      rbase = half * CHB + g * L
      ts = []
      for i in range(L):
        row = rbase + i
        t_ = None
        for j, off in enumerate(OFFS):
          x = lg_v[row, pl.ds(off, L)]
          v = avecs[j] * jnp.exp(-x) + avecs[j]
          t_ = v if t_ is None else jnp.minimum(t_, v)
        ts.append(t_)
      for k in (8, 4, 2, 1):
        pm, mk = perms[k], masks[k]
        ts = [jnp.where(mk,
                        jnp.minimum(ts[2 * j], _permute(ts[2 * j], pm)),
                        jnp.minimum(ts[2 * j + 1], _permute(ts[2 * j + 1], pm)))
              for j in range(len(ts) // 2)]
      sb_v[pl.ds(half * CHB + g * L, L)] = 1.0 / _permute(ts[0], bitrev)
      return carry

    lax.fori_loop(0, K, grp_body, 0)

    pltpu.async_copy(sb_v.at[pl.ds(half * CHB, CHB)],
                     sc_hbm.at[pl.ds(boxb, CHB)], semw)

  def pair_body(i, carry):
    t0 = 2 * i
    chunk(t0, 0, sem0, semw0, i == 0)

    @pl.when(t0 + 2 < NCHUNK)
    def _i0():
      issue_in(t0 + 2, 0, sem0)

    chunk(t0 + 1, 1, sem1, semw1, i == 0)

    @pl.when(t0 + 3 < NCHUNK)
    def _i1():
      issue_in(t0 + 3, 1, sem1)

    return carry

  lax.fori_loop(0, NCHUNK // 2, pair_body, 0)
  pltpu.make_async_copy(sb_v.at[pl.ds(0, CHB)], sc_hbm.at[pl.ds(0, CHB)],
                        semw0).wait()
  pltpu.make_async_copy(sb_v.at[pl.ds(CHB, CHB)], sc_hbm.at[pl.ds(0, CHB)],
                        semw1).wait()


_sc_scores = functools.partial(
    pl.kernel,
    out_type=jax.ShapeDtypeStruct((B * N,), jnp.float32),
    mesh=plsc.VectorSubcoreMesh(core_axis_name="c", subcore_axis_name="s",
                                num_cores=NC, num_subcores=NS),
    scratch_types=[
        pltpu.VMEM((2 * CHB, C), jnp.float32),
        pltpu.VMEM((2 * CHB,), jnp.float32),
        pltpu.VMEM((C,), jnp.float32),
        pltpu.VMEM((C,), jnp.float32),
        pltpu.SemaphoreType.DMA,
        pltpu.SemaphoreType.DMA,
        pltpu.SemaphoreType.DMA,
        pltpu.SemaphoreType.DMA,
    ],
    compiler_params=pltpu.CompilerParams(use_tc_tiling_on_sc=True))(_sc_body)


BXN = 2000


def _tc_boxes_body(ts_ref, bx_ref, out_ref):
  x = bx_ref[0]
  b = pl.program_id(0)
  hh = ts_ref[b, 0].astype(jnp.float32)
  ww = ts_ref[b, 1].astype(jnp.float32)
  r = lax.broadcasted_iota(jnp.int32, (4, 4), 0)
  c = lax.broadcasted_iota(jnp.int32, (4, 4), 1)
  m = jnp.where(r % 2 == c % 2,
                jnp.where(r < 2, 1.0, jnp.where(c < 2, -0.5, 0.5)), 0.0)
  lane = lax.broadcasted_iota(jnp.int32, (1, 4), 1)
  scale = jnp.where(lane % 2 == 0, ww, hh)
  out_ref[0] = jnp.dot(x, m, preferred_element_type=jnp.float32) * scale


def _tc_boxes(pred_boxes, target_sizes):
  return pl.pallas_call(
      _tc_boxes_body,
      grid=(B, N // BXN),
      in_specs=[
          pl.BlockSpec((B, 2), lambda b, j: (0, 0),
                       memory_space=pltpu.SMEM),
          pl.BlockSpec((1, BXN, 4), lambda b, j: (b, j, 0)),
      ],
      out_specs=pl.BlockSpec((1, BXN, 4), lambda b, j: (b, j, 0)),
      out_shape=jax.ShapeDtypeStruct((B, N, 4), jnp.float32),
  )(target_sizes, pred_boxes)


def kernel(pred_logits, pred_boxes, presence_logit_dec,
           target_sizes_boxes, target_sizes_masks):
  del target_sizes_masks
  scores_f = _sc_scores(pred_logits, presence_logit_dec)
  boxes = _tc_boxes(pred_boxes, target_sizes_boxes)
  scores = scores_f.reshape(B, N)
  labels = jnp.ones((B, N), jnp.int32)
  return scores, labels, boxes


# speedup vs baseline: 3.9968x; 3.9968x over previous
import functools

import jax
import jax.numpy as jnp
from jax import lax
from jax.experimental import pallas as pl
from jax.experimental.pallas import tpu as pltpu
from jax.experimental.pallas import tpu_sc as plsc

B, N, C = 8, 20000, 91
L = 16
NC, NS = 2, 16
NW = NC * NS
NT = -(-N // 128)
NPAD = NT * 128
UPW = -(-NT // NW)
CA, CB = 49, 42
TAIL_N0 = (NT - 1) * 128
TAIL_W = N - TAIL_N0
OFFS = (0, 16, 32, 48, 64, 75)
ASTR = 96


def _permute(g, idx):
  dn = lax.GatherDimensionNumbers(offset_dims=(), collapsed_slice_dims=(0,),
                                  start_index_map=(0,))
  return lax.gather(g, idx[:, None], dn, (1,),
                    mode=lax.GatherScatterMode.PROMISE_IN_BOUNDS)


def _sc_body(lg_hbm, tail_hbm, pr_hbm, out_hbm,
             b0_v, b1_v, acc_v, sco_v, pr_v, at_v, semA, semB, semW):
  w = lax.axis_index("s") * NC + lax.axis_index("c")

  pltpu.sync_copy(pr_hbm.at[:, :], pr_v)

  def tab_img(img, carry):
    avecs = [1.0 + jnp.exp(-pr_v[img, pl.ds(off, L)]) for off in OFFS]

    def tab_lane(l, carry2):
      bl = jnp.broadcast_to(l, (L,))
      for j, off in enumerate(OFFS):
        at_v[pl.ds((img * ASTR + off + l) * L, L)] = _permute(avecs[j], bl)
      return carry2

    lax.fori_loop(0, L, tab_lane, 0)
    return carry

  lax.fori_loop(0, B, tab_img, 0)

  def unit_tc(k):
    return jnp.minimum(w * UPW + k, NT - 1)

  def issue(tc, buf, nclass, c0, sem):
    @pl.when(tc < NT - 1)
    def _full():
      pltpu.async_copy(
          lg_hbm.at[pl.ds(c0, nclass), :, pl.ds(tc * 128, 128)], buf, sem)

    @pl.when(tc == NT - 1)
    def _tail():
      pltpu.async_copy(tail_hbm.at[pl.ds(c0, nclass), :, :], buf, sem)

  def wait_in(buf, nclass, sem):
    pltpu.make_async_copy(
        lg_hbm.at[pl.ds(0, nclass), :, pl.ds(0, 128)], buf, sem).wait()

  issue(unit_tc(0), b0_v, CA, 0, semA)
  issue(unit_tc(0), b1_v, CB, CA, semB)

  inf4 = (jnp.full((L,), jnp.inf, jnp.float32),) * 4

  def make_cbody(buf, img, cb, cbase):
    def cbody(c, accs):
      sp = at_v[pl.ds((img * ASTR + cbase + c) * L, L)]
      out = []
      for i in range(4):
        x = buf[c, img, pl.ds(cb + i * L, L)]
        out.append(jnp.minimum(accs[i], sp * jnp.exp(-x) + sp))
      return tuple(out)
    return cbody

  def compute_a(carry_unused):
    def grp(g, carry):
      img = g >> 1
      cb = (g & 1) * 64
      accs = lax.fori_loop(0, CA, make_cbody(b0_v, img, cb, 0), inf4,
                           unroll=7)
      for i in range(4):
        acc_v[pl.ds(g * 64 + i * L, L)] = accs[i]
      return carry
    lax.fori_loop(0, 16, grp, 0)

  def compute_b(p):
    def grp(g, carry):
      img = g >> 1
      cb = (g & 1) * 64
      init = tuple(acc_v[pl.ds(g * 64 + i * L, L)] for i in range(4))
      accs = lax.fori_loop(0, CB, make_cbody(b1_v, img, cb, CA), init,
                           unroll=7)
      for i in range(4):
        sco_v[p, img, pl.ds(cb + i * L, L)] = 1.0 / accs[i]
      return carry
    lax.fori_loop(0, 16, grp, 0)

  def ubody(k, carry):
    tc = unit_tc(k)
    p = k & 1
    wait_in(b0_v, CA, semA)
    compute_a(None)

    @pl.when(k + 1 < UPW)
    def _ia():
      issue(unit_tc(k + 1), b0_v, CA, 0, semA)

    wait_in(b1_v, CB, semB)

    @pl.when(k >= 2)
    def _dw():
      pltpu.make_async_copy(sco_v.at[p], out_hbm.at[:, pl.ds(0, 128)],
                            semW).wait()

    compute_b(p)
    pltpu.async_copy(sco_v.at[p], out_hbm.at[:, pl.ds(tc * 128, 128)], semW)

    @pl.when(k + 1 < UPW)
    def _ib():
      issue(unit_tc(k + 1), b1_v, CB, CA, semB)

    return carry

  lax.fori_loop(0, UPW, ubody, 0)
  for _ in range(2):
    pltpu.make_async_copy(sco_v.at[0], out_hbm.at[:, pl.ds(0, 128)],
                          semW).wait()


_sc_scores = functools.partial(
    pl.kernel,
    out_type=jax.ShapeDtypeStruct((B, NPAD), jnp.float32),
    mesh=plsc.VectorSubcoreMesh(core_axis_name="c", subcore_axis_name="s",
                                num_cores=NC, num_subcores=NS),
    scratch_types=[
        pltpu.VMEM((CA, B, 128), jnp.float32),
        pltpu.VMEM((CB, B, 128), jnp.float32),
        pltpu.VMEM((16 * 64,), jnp.float32),
        pltpu.VMEM((2, B, 128), jnp.float32),
        pltpu.VMEM((B, 128), jnp.float32),
        pltpu.VMEM((B * ASTR * L,), jnp.float32),
        pltpu.SemaphoreType.DMA,
        pltpu.SemaphoreType.DMA,
        pltpu.SemaphoreType.DMA,
    ],
    compiler_params=pltpu.CompilerParams(use_tc_tiling_on_sc=True))(_sc_body)


def _tc_boxes_body(ts_ref, bx_ref, out_ref):
  x = bx_ref[0]
  b = pl.program_id(0)
  hh = ts_ref[0, b].astype(jnp.float32)
  ww = ts_ref[1, b].astype(jnp.float32)
  row = lax.broadcasted_iota(jnp.int32, (4, N), 0)
  half = jnp.where(row >= 2, 0.5, -0.5)
  cxy = jnp.concatenate([x[0:2], x[0:2]], axis=0)
  wh = jnp.concatenate([x[2:4], x[2:4]], axis=0)
  scale = jnp.where(row % 2 == 0, ww, hh)
  out_ref[0] = (cxy + half * wh) * scale


def _tc_boxes(bxt, tst):
  return pl.pallas_call(
      _tc_boxes_body,
      grid=(B,),
      in_specs=[
          pl.BlockSpec((2, B), lambda b: (0, 0), memory_space=pltpu.SMEM),
          pl.BlockSpec((1, 4, N), lambda b: (b, 0, 0)),
      ],
      out_specs=pl.BlockSpec((1, 4, N), lambda b: (b, 0, 0)),
      out_shape=jax.ShapeDtypeStruct((B, 4, N), jnp.float32),
  )(tst, bxt)


def kernel(pred_logits, pred_boxes, presence_logit_dec,
           target_sizes_boxes, target_sizes_masks):
  del target_sizes_masks
  lgt = jnp.transpose(pred_logits, (2, 0, 1))
  bxt = jnp.transpose(pred_boxes, (0, 2, 1))
  tst = jnp.transpose(target_sizes_boxes, (1, 0))
  lg_tail = jnp.pad(lgt[:, :, TAIL_N0:], ((0, 0), (0, 0), (0, 128 - TAIL_W)))
  pr_pad = jnp.pad(presence_logit_dec, ((0, 0), (0, 128 - C)))
  scores_p = _sc_scores(lgt, lg_tail, pr_pad)
  boxes_t = _tc_boxes(bxt, tst)
  scores = scores_p[:, :N]
  labels = jnp.ones((B, N), jnp.int32)
  boxes = jnp.transpose(boxes_t, (0, 2, 1))
  return scores, labels, boxes


# speedup vs baseline: 4.0067x; 1.0025x over previous
import functools

import jax
import jax.numpy as jnp
from jax import lax
from jax.experimental import pallas as pl
from jax.experimental.pallas import tpu as pltpu
from jax.experimental.pallas import tpu_sc as plsc

B, N, C = 8, 20000, 91
L = 16
NC, NS = 2, 16
NW = NC * NS
NT = -(-N // 128)
NPAD = NT * 128
UPW = -(-NT // NW)
CA, CB = 49, 42
TAIL_N0 = (NT - 1) * 128
TAIL_W = N - TAIL_N0
OFFS = (0, 16, 32, 48, 64, 75)
ASTR = 96


def _permute(g, idx):
  dn = lax.GatherDimensionNumbers(offset_dims=(), collapsed_slice_dims=(0,),
                                  start_index_map=(0,))
  return lax.gather(g, idx[:, None], dn, (1,),
                    mode=lax.GatherScatterMode.PROMISE_IN_BOUNDS)


def _sc_body(lg_hbm, tail_hbm, pr_hbm, out_hbm,
             b0_v, b1_v, acc_v, sco_v, pr_v, at_v, semA, semB, semW):
  w = lax.axis_index("s") * NC + lax.axis_index("c")

  pltpu.sync_copy(pr_hbm.at[:, :], pr_v)

  def tab_img(img, carry):
    avecs = [1.0 + jnp.exp(-pr_v[img, pl.ds(off, L)]) for off in OFFS]

    def tab_lane(l, carry2):
      bl = jnp.broadcast_to(l, (L,))
      for j, off in enumerate(OFFS):
        at_v[pl.ds((img * ASTR + off + l) * L, L)] = _permute(avecs[j], bl)
      return carry2

    lax.fori_loop(0, L, tab_lane, 0)
    return carry

  lax.fori_loop(0, B, tab_img, 0)

  def unit_tc(k):
    return jnp.minimum(w * UPW + k, NT - 1)

  def issue(tc, buf, nclass, c0, sem):
    @pl.when(tc < NT - 1)
    def _full():
      pltpu.async_copy(
          lg_hbm.at[pl.ds(c0, nclass), :, pl.ds(tc * 128, 128)], buf, sem)

    @pl.when(tc == NT - 1)
    def _tail():
      pltpu.async_copy(tail_hbm.at[pl.ds(c0, nclass), :, :], buf, sem)

  def wait_in(buf, nclass, sem):
    pltpu.make_async_copy(
        lg_hbm.at[pl.ds(0, nclass), :, pl.ds(0, 128)], buf, sem).wait()

  issue(unit_tc(0), b0_v, CA, 0, semA)
  issue(unit_tc(0), b1_v, CB, CA, semB)

  NA = 8
  inf8 = (jnp.full((L,), jnp.inf, jnp.float32),) * NA

  def make_cbody(buf, img, cbase):
    def cbody(c, accs):
      sp = at_v[pl.ds((img * ASTR + cbase + c) * L, L)]
      out = []
      for i in range(NA):
        x = buf[c, img, pl.ds(i * L, L)]
        out.append(jnp.minimum(accs[i], sp * jnp.exp(-x) + sp))
      return tuple(out)
    return cbody

  def compute_a(carry_unused):
    def grp(img, carry):
      accs = lax.fori_loop(0, CA, make_cbody(b0_v, img, 0), inf8, unroll=7)
      for i in range(NA):
        acc_v[pl.ds(img * 128 + i * L, L)] = accs[i]
      return carry
    lax.fori_loop(0, B, grp, 0)

  def compute_b(p):
    def grp(img, carry):
      init = tuple(acc_v[pl.ds(img * 128 + i * L, L)] for i in range(NA))
      accs = lax.fori_loop(0, CB, make_cbody(b1_v, img, CA), init, unroll=7)
      for i in range(NA):
        sco_v[p, img, pl.ds(i * L, L)] = 1.0 / accs[i]
      return carry
    lax.fori_loop(0, B, grp, 0)

  def ubody(k, carry):
    tc = unit_tc(k)
    p = k & 1
    wait_in(b0_v, CA, semA)
    compute_a(None)

    @pl.when(k + 1 < UPW)
    def _ia():
      issue(unit_tc(k + 1), b0_v, CA, 0, semA)

    wait_in(b1_v, CB, semB)

    @pl.when(k >= 2)
    def _dw():
      pltpu.make_async_copy(sco_v.at[p], out_hbm.at[:, pl.ds(0, 128)],
                            semW).wait()

    compute_b(p)
    pltpu.async_copy(sco_v.at[p], out_hbm.at[:, pl.ds(tc * 128, 128)], semW)

    @pl.when(k + 1 < UPW)
    def _ib():
      issue(unit_tc(k + 1), b1_v, CB, CA, semB)

    return carry

  lax.fori_loop(0, UPW, ubody, 0)
  for _ in range(2):
    pltpu.make_async_copy(sco_v.at[0], out_hbm.at[:, pl.ds(0, 128)],
                          semW).wait()


_sc_scores = functools.partial(
    pl.kernel,
    out_type=jax.ShapeDtypeStruct((B, NPAD), jnp.float32),
    mesh=plsc.VectorSubcoreMesh(core_axis_name="c", subcore_axis_name="s",
                                num_cores=NC, num_subcores=NS),
    scratch_types=[
        pltpu.VMEM((CA, B, 128), jnp.float32),
        pltpu.VMEM((CB, B, 128), jnp.float32),
        pltpu.VMEM((16 * 64,), jnp.float32),
        pltpu.VMEM((2, B, 128), jnp.float32),
        pltpu.VMEM((B, 128), jnp.float32),
        pltpu.VMEM((B * ASTR * L,), jnp.float32),
        pltpu.SemaphoreType.DMA,
        pltpu.SemaphoreType.DMA,
        pltpu.SemaphoreType.DMA,
    ],
    compiler_params=pltpu.CompilerParams(use_tc_tiling_on_sc=True))(_sc_body)


def _tc_boxes_body(ts_ref, bx_ref, out_ref):
  x = bx_ref[0]
  b = pl.program_id(0)
  hh = ts_ref[0, b].astype(jnp.float32)
  ww = ts_ref[1, b].astype(jnp.float32)
  row = lax.broadcasted_iota(jnp.int32, (4, N), 0)
  half = jnp.where(row >= 2, 0.5, -0.5)
  cxy = jnp.concatenate([x[0:2], x[0:2]], axis=0)
  wh = jnp.concatenate([x[2:4], x[2:4]], axis=0)
  scale = jnp.where(row % 2 == 0, ww, hh)
  out_ref[0] = (cxy + half * wh) * scale


def _tc_boxes(bxt, tst):
  return pl.pallas_call(
      _tc_boxes_body,
      grid=(B,),
      in_specs=[
          pl.BlockSpec((2, B), lambda b: (0, 0), memory_space=pltpu.SMEM),
          pl.BlockSpec((1, 4, N), lambda b: (b, 0, 0)),
      ],
      out_specs=pl.BlockSpec((1, 4, N), lambda b: (b, 0, 0)),
      out_shape=jax.ShapeDtypeStruct((B, 4, N), jnp.float32),
  )(tst, bxt)


def kernel(pred_logits, pred_boxes, presence_logit_dec,
           target_sizes_boxes, target_sizes_masks):
  del target_sizes_masks
  lgt = jnp.transpose(pred_logits, (2, 0, 1))
  bxt = jnp.transpose(pred_boxes, (0, 2, 1))
  tst = jnp.transpose(target_sizes_boxes, (1, 0))
  lg_tail = jnp.pad(lgt[:, :, TAIL_N0:], ((0, 0), (0, 0), (0, 128 - TAIL_W)))
  pr_pad = jnp.pad(presence_logit_dec, ((0, 0), (0, 128 - C)))
  scores_p = _sc_scores(lgt, lg_tail, pr_pad)
  boxes_t = _tc_boxes(bxt, tst)
  scores = scores_p[:, :N]
  labels = jnp.ones((B, N), jnp.int32)
  boxes = jnp.transpose(boxes_t, (0, 2, 1))
  return scores, labels, boxes


# speedup vs baseline: 5.4525x; 1.3609x over previous
import functools

import jax
import jax.numpy as jnp
from jax import lax
from jax.experimental import pallas as pl
from jax.experimental.pallas import tpu as pltpu
from jax.experimental.pallas import tpu_sc as plsc

B, N, C = 8, 20000, 91
L = 16
NC, NS = 2, 16
NW = NC * NS
ST = 48
NSC = ST * 128
UPW = 2
CA, CB = 49, 42
OFFS = (0, 16, 32, 48, 64, 75)
ASTR = 96
BN = 2048
NTC = N - NSC


def _permute(g, idx):
  dn = lax.GatherDimensionNumbers(offset_dims=(), collapsed_slice_dims=(0,),
                                  start_index_map=(0,))
  return lax.gather(g, idx[:, None], dn, (1,),
                    mode=lax.GatherScatterMode.PROMISE_IN_BOUNDS)


def _sc_body(lg_hbm, pr_hbm, out_hbm,
             b0_v, b1_v, acc_v, sco_v, pr_v, at_v, semA, semB, semW):
  w = lax.axis_index("s") * NC + lax.axis_index("c")
  active = w * UPW < ST

  pltpu.sync_copy(pr_hbm.at[:, :], pr_v)

  def tab_img(img, carry):
    avecs = [1.0 + jnp.exp(-pr_v[img, pl.ds(off, L)]) for off in OFFS]

    def tab_lane(l, carry2):
      bl = jnp.broadcast_to(l, (L,))
      for j, off in enumerate(OFFS):
        at_v[pl.ds((img * ASTR + off + l) * L, L)] = _permute(avecs[j], bl)
      return carry2

    lax.fori_loop(0, L, tab_lane, 0)
    return carry

  lax.fori_loop(0, B, tab_img, 0)

  def issue(tc, buf, nclass, c0, sem):
    pltpu.async_copy(
        lg_hbm.at[pl.ds(c0, nclass), :, pl.ds(tc * 128, 128)], buf, sem)

  def wait_in(buf, nclass, sem):
    pltpu.make_async_copy(
        lg_hbm.at[pl.ds(0, nclass), :, pl.ds(0, 128)], buf, sem).wait()

  @pl.when(active)
  def _prologue():
    issue(w * UPW, b0_v, CA, 0, semA)
    issue(w * UPW, b1_v, CB, CA, semB)

  NA = 8
  inf8 = (jnp.full((L,), jnp.inf, jnp.float32),) * NA

  def make_cbody(buf, img, cbase):
    def cbody(c, accs):
      sp = at_v[pl.ds((img * ASTR + cbase + c) * L, L)]
      out = []
      for i in range(NA):
        x = buf[c, img, pl.ds(i * L, L)]
        out.append(jnp.minimum(accs[i], sp * jnp.exp(-x) + sp))
      return tuple(out)
    return cbody

  def compute_a(carry_unused):
    def grp(img, carry):
      accs = lax.fori_loop(0, CA, make_cbody(b0_v, img, 0), inf8, unroll=7)
      for i in range(NA):
        acc_v[pl.ds(img * 128 + i * L, L)] = accs[i]
      return carry
    lax.fori_loop(0, B, grp, 0)

  def compute_b(p):
    def grp(img, carry):
      init = tuple(acc_v[pl.ds(img * 128 + i * L, L)] for i in range(NA))
      accs = lax.fori_loop(0, CB, make_cbody(b1_v, img, CA), init, unroll=7)
      for i in range(NA):
        sco_v[p, img, pl.ds(i * L, L)] = 1.0 / accs[i]
      return carry
    lax.fori_loop(0, B, grp, 0)

  def ubody(k, carry):
    tc = w * UPW + k
    p = k & 1
    wait_in(b0_v, CA, semA)
    compute_a(None)

    @pl.when(k + 1 < UPW)
    def _ia():
      issue(tc + 1, b0_v, CA, 0, semA)

    wait_in(b1_v, CB, semB)
    compute_b(p)
    pltpu.async_copy(sco_v.at[p], out_hbm.at[:, pl.ds(tc * 128, 128)], semW)

    @pl.when(k + 1 < UPW)
    def _ib():
      issue(tc + 1, b1_v, CB, CA, semB)

    return carry

  @pl.when(active)
  def _run():
    lax.fori_loop(0, UPW, ubody, 0)
    for _ in range(UPW):
      pltpu.make_async_copy(sco_v.at[0], out_hbm.at[:, pl.ds(0, 128)],
                            semW).wait()


_sc_scores = functools.partial(
    pl.kernel,
    out_type=jax.ShapeDtypeStruct((B, NSC), jnp.float32),
    mesh=plsc.VectorSubcoreMesh(core_axis_name="c", subcore_axis_name="s",
                                num_cores=NC, num_subcores=NS),
    scratch_types=[
        pltpu.VMEM((CA, B, 128), jnp.float32),
        pltpu.VMEM((CB, B, 128), jnp.float32),
        pltpu.VMEM((B * 128,), jnp.float32),
        pltpu.VMEM((2, B, 128), jnp.float32),
        pltpu.VMEM((B, 128), jnp.float32),
        pltpu.VMEM((B * ASTR * L,), jnp.float32),
        pltpu.SemaphoreType.DMA,
        pltpu.SemaphoreType.DMA,
        pltpu.SemaphoreType.DMA,
    ],
    compiler_params=pltpu.CompilerParams(use_tc_tiling_on_sc=True))(_sc_body)


def _tc_scores_body(prt_ref, lg_ref, out_ref):
  x = lg_ref[...]
  a = 1.0 + jnp.exp(-prt_ref[...][:C])
  acc = jnp.min(a[:, :, None] * jnp.exp(-x) + a[:, :, None], axis=0)
  out_ref[...] = 1.0 / acc


def _tc_scores(lgt, prt):
  return pl.pallas_call(
      _tc_scores_body,
      grid=(NTC // BN + 1,),
      in_specs=[
          pl.BlockSpec((128, B), lambda j: (0, 0)),
          pl.BlockSpec((C, B, BN), lambda j: (0, 0, j + NSC // BN)),
      ],
      out_specs=pl.BlockSpec((B, BN), lambda j: (0, j)),
      out_shape=jax.ShapeDtypeStruct((B, NTC), jnp.float32),
  )(prt, lgt)


def _tc_boxes_body(ts_ref, bx_ref, out_ref):
  x = bx_ref[0]
  b = pl.program_id(0)
  hh = ts_ref[0, b].astype(jnp.float32)
  ww = ts_ref[1, b].astype(jnp.float32)
  row = lax.broadcasted_iota(jnp.int32, (4, N), 0)
  half = jnp.where(row >= 2, 0.5, -0.5)
  cxy = jnp.concatenate([x[0:2], x[0:2]], axis=0)
  wh = jnp.concatenate([x[2:4], x[2:4]], axis=0)
  scale = jnp.where(row % 2 == 0, ww, hh)
  out_ref[0] = (cxy + half * wh) * scale


def _tc_boxes(bxt, tst):
  return pl.pallas_call(
      _tc_boxes_body,
      grid=(B,),
      in_specs=[
          pl.BlockSpec((2, B), lambda b: (0, 0), memory_space=pltpu.SMEM),
          pl.BlockSpec((1, 4, N), lambda b: (b, 0, 0)),
      ],
      out_specs=pl.BlockSpec((1, 4, N), lambda b: (b, 0, 0)),
      out_shape=jax.ShapeDtypeStruct((B, 4, N), jnp.float32),
  )(tst, bxt)


def kernel(pred_logits, pred_boxes, presence_logit_dec,
           target_sizes_boxes, target_sizes_masks):
  del target_sizes_masks
  lgt = jnp.transpose(pred_logits, (2, 0, 1))
  bxt = jnp.transpose(pred_boxes, (0, 2, 1))
  tst = jnp.transpose(target_sizes_boxes, (1, 0))
  pr_pad = jnp.pad(presence_logit_dec, ((0, 0), (0, 128 - C)))
  prt = jnp.transpose(pr_pad, (1, 0))
  sc_part = _sc_scores(lgt, pr_pad)
  tc_part = _tc_scores(lgt, prt)
  boxes_t = _tc_boxes(bxt, tst)
  scores = jnp.concatenate([sc_part, tc_part], axis=1)
  labels = jnp.ones((B, N), jnp.int32)
  boxes = jnp.transpose(boxes_t, (0, 2, 1))
  return scores, labels, boxes


# speedup vs baseline: 5.5715x; 1.0218x over previous
import functools

import jax
import jax.numpy as jnp
from jax import lax
from jax.experimental import pallas as pl
from jax.experimental.pallas import tpu as pltpu
from jax.experimental.pallas import tpu_sc as plsc

B, N, C = 8, 20000, 91
L = 16
NC, NS = 2, 16
NW = NC * NS
ST = 48
NSC = ST * 128
UPW = 2
CA, CB = 49, 42
OFFS = (0, 16, 32, 48, 64, 75)
ASTR = 96
BN = 2048
NTC = N - NSC


def _permute(g, idx):
  dn = lax.GatherDimensionNumbers(offset_dims=(), collapsed_slice_dims=(0,),
                                  start_index_map=(0,))
  return lax.gather(g, idx[:, None], dn, (1,),
                    mode=lax.GatherScatterMode.PROMISE_IN_BOUNDS)


def _sc_body(lg_hbm, pr_hbm, out_hbm,
             b0_v, b1_v, acc_v, sco_v, pr_v, at_v, semA, semB, semW):
  w = lax.axis_index("s") * NC + lax.axis_index("c")
  active = w * UPW < ST

  pltpu.sync_copy(pr_hbm.at[:, :], pr_v)

  def tab_img(img, carry):
    avecs = [1.0 + jnp.exp(-pr_v[img, pl.ds(off, L)]) for off in OFFS]

    def tab_lane(l, carry2):
      bl = jnp.broadcast_to(l, (L,))
      for j, off in enumerate(OFFS):
        at_v[pl.ds((img * ASTR + off + l) * L, L)] = _permute(avecs[j], bl)
      return carry2

    lax.fori_loop(0, L, tab_lane, 0)
    return carry

  lax.fori_loop(0, B, tab_img, 0)

  def issue(tc, buf, nclass, c0, sem):
    pltpu.async_copy(
        lg_hbm.at[pl.ds(c0, nclass), :, pl.ds(tc * 128, 128)], buf, sem)

  def wait_in(buf, nclass, sem):
    pltpu.make_async_copy(
        lg_hbm.at[pl.ds(0, nclass), :, pl.ds(0, 128)], buf, sem).wait()

  @pl.when(active)
  def _prologue():
    issue(w * UPW, b0_v, CA, 0, semA)
    issue(w * UPW, b1_v, CB, CA, semB)

  NA = 8
  inf8 = (jnp.full((L,), jnp.inf, jnp.float32),) * NA

  def make_cbody(buf, img, cbase):
    def cbody(c, accs):
      sp = at_v[pl.ds((img * ASTR + cbase + c) * L, L)]
      out = []
      for i in range(NA):
        x = buf[c, img, pl.ds(i * L, L)]
        out.append(jnp.minimum(accs[i], sp * jnp.exp(-x) + sp))
      return tuple(out)
    return cbody

  def compute_a(carry_unused):
    def grp(img, carry):
      accs = lax.fori_loop(0, CA, make_cbody(b0_v, img, 0), inf8, unroll=7)
      for i in range(NA):
        acc_v[pl.ds(img * 128 + i * L, L)] = accs[i]
      return carry
    lax.fori_loop(0, B, grp, 0)

  def compute_b(p):
    def grp(img, carry):
      init = tuple(acc_v[pl.ds(img * 128 + i * L, L)] for i in range(NA))
      accs = lax.fori_loop(0, CB, make_cbody(b1_v, img, CA), init, unroll=7)
      for i in range(NA):
        sco_v[p, img, pl.ds(i * L, L)] = 1.0 / accs[i]
      return carry
    lax.fori_loop(0, B, grp, 0)

  def ubody(k, carry):
    tc = w * UPW + k
    p = k & 1
    wait_in(b0_v, CA, semA)
    compute_a(None)

    @pl.when(k + 1 < UPW)
    def _ia():
      issue(tc + 1, b0_v, CA, 0, semA)

    wait_in(b1_v, CB, semB)
    compute_b(p)
    pltpu.async_copy(sco_v.at[p], out_hbm.at[:, pl.ds(tc * 128, 128)], semW)

    @pl.when(k + 1 < UPW)
    def _ib():
      issue(tc + 1, b1_v, CB, CA, semB)

    return carry

  @pl.when(active)
  def _run():
    lax.fori_loop(0, UPW, ubody, 0)
    for _ in range(UPW):
      pltpu.make_async_copy(sco_v.at[0], out_hbm.at[:, pl.ds(0, 128)],
                            semW).wait()


_sc_scores = functools.partial(
    pl.kernel,
    out_type=jax.ShapeDtypeStruct((B, NSC), jnp.float32),
    mesh=plsc.VectorSubcoreMesh(core_axis_name="c", subcore_axis_name="s",
                                num_cores=NC, num_subcores=NS),
    scratch_types=[
        pltpu.VMEM((CA, B, 128), jnp.float32),
        pltpu.VMEM((CB, B, 128), jnp.float32),
        pltpu.VMEM((B * 128,), jnp.float32),
        pltpu.VMEM((2, B, 128), jnp.float32),
        pltpu.VMEM((B, 128), jnp.float32),
        pltpu.VMEM((B * ASTR * L,), jnp.float32),
        pltpu.SemaphoreType.DMA,
        pltpu.SemaphoreType.DMA,
        pltpu.SemaphoreType.DMA,
    ],
    compiler_params=pltpu.CompilerParams(use_tc_tiling_on_sc=True))(_sc_body)


NG = NTC // BN + 1
BB = 2944


def _tc_main_body(ts_ref, prt_ref, lg_ref, bx_ref,
                  sco_ref, box_ref, lab_ref):
  x = lg_ref[...]
  a = 1.0 + jnp.exp(-prt_ref[...][:C])
  acc = jnp.min(a[:, :, None] * jnp.exp(-x) + a[:, :, None], axis=0)
  sco_ref[...] = 1.0 / acc
  xb = bx_ref[...]
  ts = ts_ref[...].astype(jnp.float32)
  hh = ts[0][:, None, None]
  ww = ts[1][:, None, None]
  coord = lax.broadcasted_iota(jnp.int32, (B, 4, BB), 1)
  half = jnp.where(coord >= 2, 0.5, -0.5)
  cxy = jnp.concatenate([xb[:, 0:2], xb[:, 0:2]], axis=1)
  wh = jnp.concatenate([xb[:, 2:4], xb[:, 2:4]], axis=1)
  scale = jnp.where(coord % 2 == 0, ww, hh)
  box_ref[...] = (cxy + half * wh) * scale
  lab_ref[...] = jnp.ones((B, BB), jnp.int32)


def _tc_main(lgt, prt, bxt, tst):
  return pl.pallas_call(
      _tc_main_body,
      grid=(NG,),
      in_specs=[
          pl.BlockSpec((2, B), lambda j: (0, 0)),
          pl.BlockSpec((128, B), lambda j: (0, 0)),
          pl.BlockSpec((C, B, BN), lambda j: (0, 0, j + NSC // BN)),
          pl.BlockSpec((B, 4, BB), lambda j: (0, 0, j)),
      ],
      out_specs=[
          pl.BlockSpec((B, BN), lambda j: (0, j)),
          pl.BlockSpec((B, 4, BB), lambda j: (0, 0, j)),
          pl.BlockSpec((B, BB), lambda j: (0, j)),
      ],
      out_shape=[
          jax.ShapeDtypeStruct((B, NTC), jnp.float32),
          jax.ShapeDtypeStruct((B, 4, N), jnp.float32),
          jax.ShapeDtypeStruct((B, N), jnp.int32),
      ],
  )(tst, prt, lgt, bxt)


def kernel(pred_logits, pred_boxes, presence_logit_dec,
           target_sizes_boxes, target_sizes_masks):
  del target_sizes_masks
  lgt = jnp.transpose(pred_logits, (2, 0, 1))
  bxt = jnp.transpose(pred_boxes, (0, 2, 1))
  tst = jnp.transpose(target_sizes_boxes, (1, 0))
  pr_pad = jnp.pad(presence_logit_dec, ((0, 0), (0, 128 - C)))
  prt = jnp.transpose(pr_pad, (1, 0))
  sc_part = _sc_scores(lgt, pr_pad)
  tc_part, boxes_t, labels = _tc_main(lgt, prt, bxt, tst)
  scores = jnp.concatenate([sc_part, tc_part], axis=1)
  boxes = jnp.transpose(boxes_t, (0, 2, 1))
  return scores, labels, boxes


# speedup vs baseline: 5.6338x; 1.0112x over previous
import functools

import jax
import jax.numpy as jnp
from jax import lax
from jax.experimental import pallas as pl
from jax.experimental.pallas import tpu as pltpu
from jax.experimental.pallas import tpu_sc as plsc

B, N, C = 8, 20000, 91
L = 16
NC, NS = 2, 16
NW = NC * NS
ST = 40
NSC = ST * 128
UPW = 2
CA, CB = 49, 42
OFFS = (0, 16, 32, 48, 64, 75)
ASTR = 96
BN = 1024
NTC = N - NSC


def _permute(g, idx):
  dn = lax.GatherDimensionNumbers(offset_dims=(), collapsed_slice_dims=(0,),
                                  start_index_map=(0,))
  return lax.gather(g, idx[:, None], dn, (1,),
                    mode=lax.GatherScatterMode.PROMISE_IN_BOUNDS)


def _sc_body(lg_hbm, pr_hbm, out_hbm,
             b0_v, b1_v, acc_v, sco_v, pr_v, at_v, semA, semB, semW):
  w = lax.axis_index("s") * NC + lax.axis_index("c")
  active = w * UPW < ST

  pltpu.sync_copy(pr_hbm.at[:, :], pr_v)

  def tab_img(img, carry):
    avecs = [1.0 + jnp.exp(-pr_v[img, pl.ds(off, L)]) for off in OFFS]

    def tab_lane(l, carry2):
      bl = jnp.broadcast_to(l, (L,))
      for j, off in enumerate(OFFS):
        at_v[pl.ds((img * ASTR + off + l) * L, L)] = _permute(avecs[j], bl)
      return carry2

    lax.fori_loop(0, L, tab_lane, 0)
    return carry

  lax.fori_loop(0, B, tab_img, 0)

  def issue(tc, buf, nclass, c0, sem):
    pltpu.async_copy(
        lg_hbm.at[pl.ds(c0, nclass), :, pl.ds(tc * 128, 128)], buf, sem)

  def wait_in(buf, nclass, sem):
    pltpu.make_async_copy(
        lg_hbm.at[pl.ds(0, nclass), :, pl.ds(0, 128)], buf, sem).wait()

  @pl.when(active)
  def _prologue():
    issue(w * UPW, b0_v, CA, 0, semA)
    issue(w * UPW, b1_v, CB, CA, semB)

  NA = 8
  inf8 = (jnp.full((L,), jnp.inf, jnp.float32),) * NA

  def make_cbody(buf, img, cbase):
    def cbody(c, accs):
      sp = at_v[pl.ds((img * ASTR + cbase + c) * L, L)]
      out = []
      for i in range(NA):
        x = buf[c, img, pl.ds(i * L, L)]
        out.append(jnp.minimum(accs[i], sp * jnp.exp(-x) + sp))
      return tuple(out)
    return cbody

  def compute_a(carry_unused):
    def grp(img, carry):
      accs = lax.fori_loop(0, CA, make_cbody(b0_v, img, 0), inf8, unroll=7)
      for i in range(NA):
        acc_v[pl.ds(img * 128 + i * L, L)] = accs[i]
      return carry
    lax.fori_loop(0, B, grp, 0)

  def compute_b(p):
    def grp(img, carry):
      init = tuple(acc_v[pl.ds(img * 128 + i * L, L)] for i in range(NA))
      accs = lax.fori_loop(0, CB, make_cbody(b1_v, img, CA), init, unroll=7)
      for i in range(NA):
        sco_v[p, img, pl.ds(i * L, L)] = 1.0 / accs[i]
      return carry
    lax.fori_loop(0, B, grp, 0)

  def ubody(k, carry):
    tc = w * UPW + k
    p = k & 1
    wait_in(b0_v, CA, semA)
    compute_a(None)

    @pl.when(k + 1 < UPW)
    def _ia():
      issue(tc + 1, b0_v, CA, 0, semA)

    wait_in(b1_v, CB, semB)
    compute_b(p)
    pltpu.async_copy(sco_v.at[p], out_hbm.at[:, pl.ds(tc * 128, 128)], semW)

    @pl.when(k + 1 < UPW)
    def _ib():
      issue(tc + 1, b1_v, CB, CA, semB)

    return carry

  @pl.when(active)
  def _run():
    lax.fori_loop(0, UPW, ubody, 0)
    for _ in range(UPW):
      pltpu.make_async_copy(sco_v.at[0], out_hbm.at[:, pl.ds(0, 128)],
                            semW).wait()


_sc_scores = functools.partial(
    pl.kernel,
    out_type=jax.ShapeDtypeStruct((B, NSC), jnp.float32),
    mesh=plsc.VectorSubcoreMesh(core_axis_name="c", subcore_axis_name="s",
                                num_cores=NC, num_subcores=NS),
    scratch_types=[
        pltpu.VMEM((CA, B, 128), jnp.float32),
        pltpu.VMEM((CB, B, 128), jnp.float32),
        pltpu.VMEM((B * 128,), jnp.float32),
        pltpu.VMEM((2, B, 128), jnp.float32),
        pltpu.VMEM((B, 128), jnp.float32),
        pltpu.VMEM((B * ASTR * L,), jnp.float32),
        pltpu.SemaphoreType.DMA,
        pltpu.SemaphoreType.DMA,
        pltpu.SemaphoreType.DMA,
    ],
    compiler_params=pltpu.CompilerParams(use_tc_tiling_on_sc=True))(_sc_body)


NG = NTC // BN + 1
BB = 128 * (-(-N // (128 * NG)))


def _tc_main_body(ts_ref, prt_ref, lg_ref, bx_ref,
                  sco_ref, box_ref, lab_ref):
  x = lg_ref[...]
  a = 1.0 + jnp.exp(-prt_ref[...][:C])
  acc = jnp.min(a[:, :, None] * jnp.exp(-x) + a[:, :, None], axis=0)
  sco_ref[...] = 1.0 / acc
  xb = bx_ref[...]
  ts = ts_ref[...].astype(jnp.float32)
  hh = ts[0][:, None, None]
  ww = ts[1][:, None, None]
  coord = lax.broadcasted_iota(jnp.int32, (B, 4, BB), 1)
  half = jnp.where(coord >= 2, 0.5, -0.5)
  cxy = jnp.concatenate([xb[:, 0:2], xb[:, 0:2]], axis=1)
  wh = jnp.concatenate([xb[:, 2:4], xb[:, 2:4]], axis=1)
  scale = jnp.where(coord % 2 == 0, ww, hh)
  box_ref[...] = (cxy + half * wh) * scale
  lab_ref[...] = jnp.ones((B, BB), jnp.int32)


def _tc_main(lgt, prt, bxt, tst):
  return pl.pallas_call(
      _tc_main_body,
      grid=(NG,),
      in_specs=[
          pl.BlockSpec((2, B), lambda j: (0, 0)),
          pl.BlockSpec((128, B), lambda j: (0, 0)),
          pl.BlockSpec((C, B, BN), lambda j: (0, 0, j + NSC // BN)),
          pl.BlockSpec((B, 4, BB), lambda j: (0, 0, j)),
      ],
      out_specs=[
          pl.BlockSpec((B, BN), lambda j: (0, j)),
          pl.BlockSpec((B, 4, BB), lambda j: (0, 0, j)),
          pl.BlockSpec((B, BB), lambda j: (0, j)),
      ],
      out_shape=[
          jax.ShapeDtypeStruct((B, NTC), jnp.float32),
          jax.ShapeDtypeStruct((B, 4, N), jnp.float32),
          jax.ShapeDtypeStruct((B, N), jnp.int32),
      ],
  )(tst, prt, lgt, bxt)


def kernel(pred_logits, pred_boxes, presence_logit_dec,
           target_sizes_boxes, target_sizes_masks):
  del target_sizes_masks
  lgt = jnp.transpose(pred_logits, (2, 0, 1))
  bxt = jnp.transpose(pred_boxes, (0, 2, 1))
  tst = jnp.transpose(target_sizes_boxes, (1, 0))
  pr_pad = jnp.pad(presence_logit_dec, ((0, 0), (0, 128 - C)))
  prt = jnp.transpose(pr_pad, (1, 0))
  sc_part = _sc_scores(lgt, pr_pad)
  tc_part, boxes_t, labels = _tc_main(lgt, prt, bxt, tst)
  scores = jnp.concatenate([sc_part, tc_part], axis=1)
  boxes = jnp.transpose(boxes_t, (0, 2, 1))
  return scores, labels, boxes


# speedup vs baseline: 5.8773x; 1.0432x over previous
import functools

import jax
import jax.numpy as jnp
from jax import lax
from jax.experimental import pallas as pl
from jax.experimental.pallas import tpu as pltpu
from jax.experimental.pallas import tpu_sc as plsc

B, N, C = 8, 20000, 91
L = 16
NC, NS = 2, 16
NW = NC * NS
ST = 40
NSC = ST * 128
UPW = 2
CA, CB = 49, 42
OFFS = (0, 16, 32, 48, 64, 75)
ASTR = 96
BN = 1024
NTC = N - NSC


def _permute(g, idx):
  dn = lax.GatherDimensionNumbers(offset_dims=(), collapsed_slice_dims=(0,),
                                  start_index_map=(0,))
  return lax.gather(g, idx[:, None], dn, (1,),
                    mode=lax.GatherScatterMode.PROMISE_IN_BOUNDS)


def _sc_body(lg_hbm, pr_hbm, out_hbm,
             b0_v, b1_v, acc_v, sco_v, pr_v, at_v, semA, semB, semW):
  w = lax.axis_index("s") * NC + lax.axis_index("c")
  active = w * UPW < ST

  pltpu.sync_copy(pr_hbm.at[:, :], pr_v)

  def tab_img(img, carry):
    avecs = [1.0 + jnp.exp(-pr_v[img, pl.ds(off, L)]) for off in OFFS]

    def tab_lane(l, carry2):
      bl = jnp.broadcast_to(l, (L,))
      for j, off in enumerate(OFFS):
        at_v[pl.ds((img * ASTR + off + l) * L, L)] = _permute(avecs[j], bl)
      return carry2

    lax.fori_loop(0, L, tab_lane, 0)
    return carry

  lax.fori_loop(0, B, tab_img, 0)

  def issue(tc, buf, nclass, c0, sem):
    pltpu.async_copy(
        lg_hbm.at[pl.ds(c0, nclass), :, pl.ds(tc * 128, 128)], buf, sem)

  def wait_in(buf, nclass, sem):
    pltpu.make_async_copy(
        lg_hbm.at[pl.ds(0, nclass), :, pl.ds(0, 128)], buf, sem).wait()

  @pl.when(active)
  def _prologue():
    issue(w * UPW, b0_v, CA, 0, semA)
    issue(w * UPW, b1_v, CB, CA, semB)

  NA = 8
  inf8 = (jnp.full((L,), jnp.inf, jnp.float32),) * NA

  def make_cbody(buf, img, cbase):
    def cbody(c, accs):
      sp = at_v[pl.ds((img * ASTR + cbase + c) * L, L)]
      out = []
      for i in range(NA):
        x = buf[c, img, pl.ds(i * L, L)]
        out.append(jnp.minimum(accs[i], sp * jnp.exp(-x) + sp))
      return tuple(out)
    return cbody

  def compute_a(carry_unused):
    def grp(img, carry):
      accs = lax.fori_loop(0, CA, make_cbody(b0_v, img, 0), inf8, unroll=3)
      for i in range(NA):
        acc_v[pl.ds(img * 128 + i * L, L)] = accs[i]
      return carry
    lax.fori_loop(0, B, grp, 0)

  def compute_b(p):
    def grp(img, carry):
      init = tuple(acc_v[pl.ds(img * 128 + i * L, L)] for i in range(NA))
      accs = lax.fori_loop(0, CB, make_cbody(b1_v, img, CA), init, unroll=3)
      for i in range(NA):
        sco_v[p, img, pl.ds(i * L, L)] = 1.0 / accs[i]
      return carry
    lax.fori_loop(0, B, grp, 0)

  def ubody(k, carry):
    tc = w * UPW + k
    p = k & 1
    wait_in(b0_v, CA, semA)
    compute_a(None)

    @pl.when(k + 1 < UPW)
    def _ia():
      issue(tc + 1, b0_v, CA, 0, semA)

    wait_in(b1_v, CB, semB)
    compute_b(p)
    pltpu.async_copy(sco_v.at[p], out_hbm.at[:, pl.ds(tc * 128, 128)], semW)

    @pl.when(k + 1 < UPW)
    def _ib():
      issue(tc + 1, b1_v, CB, CA, semB)

    return carry

  @pl.when(active)
  def _run():
    lax.fori_loop(0, UPW, ubody, 0)
    for _ in range(UPW):
      pltpu.make_async_copy(sco_v.at[0], out_hbm.at[:, pl.ds(0, 128)],
                            semW).wait()


_sc_scores = functools.partial(
    pl.kernel,
    out_type=jax.ShapeDtypeStruct((B, NSC), jnp.float32),
    mesh=plsc.VectorSubcoreMesh(core_axis_name="c", subcore_axis_name="s",
                                num_cores=NC, num_subcores=NS),
    scratch_types=[
        pltpu.VMEM((CA, B, 128), jnp.float32),
        pltpu.VMEM((CB, B, 128), jnp.float32),
        pltpu.VMEM((B * 128,), jnp.float32),
        pltpu.VMEM((2, B, 128), jnp.float32),
        pltpu.VMEM((B, 128), jnp.float32),
        pltpu.VMEM((B * ASTR * L,), jnp.float32),
        pltpu.SemaphoreType.DMA,
        pltpu.SemaphoreType.DMA,
        pltpu.SemaphoreType.DMA,
    ],
    compiler_params=pltpu.CompilerParams(use_tc_tiling_on_sc=True))(_sc_body)


NG = NTC // BN + 1
BB = 128 * (-(-N // (128 * NG)))


def _tc_main_body(ts_ref, prt_ref, lg_ref, bx_ref,
                  sco_ref, box_ref, lab_ref):
  x = lg_ref[...]
  a = 1.0 + jnp.exp(-prt_ref[...][:C])
  acc = jnp.min(a[:, :, None] * jnp.exp(-x) + a[:, :, None], axis=0)
  sco_ref[...] = 1.0 / acc
  xb = bx_ref[...]
  ts = ts_ref[...].astype(jnp.float32)
  hh = ts[0][:, None, None]
  ww = ts[1][:, None, None]
  coord = lax.broadcasted_iota(jnp.int32, (B, 4, BB), 1)
  half = jnp.where(coord >= 2, 0.5, -0.5)
  cxy = jnp.concatenate([xb[:, 0:2], xb[:, 0:2]], axis=1)
  wh = jnp.concatenate([xb[:, 2:4], xb[:, 2:4]], axis=1)
  scale = jnp.where(coord % 2 == 0, ww, hh)
  box_ref[...] = (cxy + half * wh) * scale
  lab_ref[...] = jnp.ones((B, BB), jnp.int32)


def _tc_main(lgt, prt, bxt, tst):
  return pl.pallas_call(
      _tc_main_body,
      grid=(NG,),
      in_specs=[
          pl.BlockSpec((2, B), lambda j: (0, 0)),
          pl.BlockSpec((128, B), lambda j: (0, 0)),
          pl.BlockSpec((C, B, BN), lambda j: (0, 0, j + NSC // BN)),
          pl.BlockSpec((B, 4, BB), lambda j: (0, 0, j)),
      ],
      out_specs=[
          pl.BlockSpec((B, BN), lambda j: (0, j)),
          pl.BlockSpec((B, 4, BB), lambda j: (0, 0, j)),
          pl.BlockSpec((B, BB), lambda j: (0, j)),
      ],
      out_shape=[
          jax.ShapeDtypeStruct((B, NTC), jnp.float32),
          jax.ShapeDtypeStruct((B, 4, N), jnp.float32),
          jax.ShapeDtypeStruct((B, N), jnp.int32),
      ],
  )(tst, prt, lgt, bxt)


def kernel(pred_logits, pred_boxes, presence_logit_dec,
           target_sizes_boxes, target_sizes_masks):
  del target_sizes_masks
  lgt = jnp.transpose(pred_logits, (2, 0, 1))
  bxt = jnp.transpose(pred_boxes, (0, 2, 1))
  tst = jnp.transpose(target_sizes_boxes, (1, 0))
  pr_pad = jnp.pad(presence_logit_dec, ((0, 0), (0, 128 - C)))
  prt = jnp.transpose(pr_pad, (1, 0))
  sc_part = _sc_scores(lgt, pr_pad)
  tc_part, boxes_t, labels = _tc_main(lgt, prt, bxt, tst)
  scores = jnp.concatenate([sc_part, tc_part], axis=1)
  boxes = jnp.transpose(boxes_t, (0, 2, 1))
  return scores, labels, boxes


# speedup vs baseline: 5.9039x; 1.0045x over previous
import functools

import jax
import jax.numpy as jnp
from jax import lax
from jax.experimental import pallas as pl
from jax.experimental.pallas import tpu as pltpu
from jax.experimental.pallas import tpu_sc as plsc

B, N, C = 8, 20000, 91
L = 16
NC, NS = 2, 16
NW = NC * NS
ST = 40
NSC = ST * 128
UPW = 2
CA, CB = 49, 42
OFFS = (0, 16, 32, 48, 64, 75)
ASTR = 96
BN = 1024
NTC = N - NSC


def _permute(g, idx):
  dn = lax.GatherDimensionNumbers(offset_dims=(), collapsed_slice_dims=(0,),
                                  start_index_map=(0,))
  return lax.gather(g, idx[:, None], dn, (1,),
                    mode=lax.GatherScatterMode.PROMISE_IN_BOUNDS)


def _sc_body(lg_hbm, pr_hbm, out_hbm,
             b0_v, b1_v, acc_v, sco_v, pr_v, at_v, semA, semB, semW):
  w = lax.axis_index("s") * NC + lax.axis_index("c")
  active = w * UPW < ST

  pltpu.sync_copy(pr_hbm.at[:, :], pr_v)

  def tab_img(img, carry):
    avecs = [1.0 + jnp.exp(-pr_v[img, pl.ds(off, L)]) for off in OFFS]

    def tab_lane(l, carry2):
      bl = jnp.broadcast_to(l, (L,))
      for j, off in enumerate(OFFS):
        at_v[pl.ds((img * ASTR + off + l) * L, L)] = _permute(avecs[j], bl)
      return carry2

    lax.fori_loop(0, L, tab_lane, 0)
    return carry

  lax.fori_loop(0, B, tab_img, 0)

  def issue(tc, buf, nclass, c0, sem):
    pltpu.async_copy(
        lg_hbm.at[pl.ds(c0, nclass), :, pl.ds(tc * 128, 128)], buf, sem)

  def wait_in(buf, nclass, sem):
    pltpu.make_async_copy(
        lg_hbm.at[pl.ds(0, nclass), :, pl.ds(0, 128)], buf, sem).wait()

  @pl.when(active)
  def _prologue():
    issue(w * UPW, b0_v, CA, 0, semA)
    issue(w * UPW, b1_v, CB, CA, semB)

  NA = 8
  inf8 = (jnp.full((L,), jnp.inf, jnp.float32),) * NA

  def make_cbody(buf, img, cbase):
    def cbody(c, accs):
      sp = at_v[pl.ds((img * ASTR + cbase + c) * L, L)]
      out = []
      for i in range(NA):
        x = buf[c, img, pl.ds(i * L, L)]
        out.append(jnp.minimum(accs[i], sp * jnp.exp(-x) + sp))
      return tuple(out)
    return cbody

  def compute_a(carry_unused):
    def grp(img, carry):
      accs = lax.fori_loop(0, CA, make_cbody(b0_v, img, 0), inf8, unroll=1)
      for i in range(NA):
        acc_v[pl.ds(img * 128 + i * L, L)] = accs[i]
      return carry
    lax.fori_loop(0, B, grp, 0)

  def compute_b(p):
    def grp(img, carry):
      init = tuple(acc_v[pl.ds(img * 128 + i * L, L)] for i in range(NA))
      accs = lax.fori_loop(0, CB, make_cbody(b1_v, img, CA), init, unroll=1)
      for i in range(NA):
        sco_v[p, img, pl.ds(i * L, L)] = 1.0 / accs[i]
      return carry
    lax.fori_loop(0, B, grp, 0)

  def ubody(k, carry):
    tc = w * UPW + k
    p = k & 1
    wait_in(b0_v, CA, semA)
    compute_a(None)

    @pl.when(k + 1 < UPW)
    def _ia():
      issue(tc + 1, b0_v, CA, 0, semA)

    wait_in(b1_v, CB, semB)
    compute_b(p)
    pltpu.async_copy(sco_v.at[p], out_hbm.at[:, pl.ds(tc * 128, 128)], semW)

    @pl.when(k + 1 < UPW)
    def _ib():
      issue(tc + 1, b1_v, CB, CA, semB)

    return carry

  @pl.when(active)
  def _run():
    lax.fori_loop(0, UPW, ubody, 0)
    for _ in range(UPW):
      pltpu.make_async_copy(sco_v.at[0], out_hbm.at[:, pl.ds(0, 128)],
                            semW).wait()


_sc_scores = functools.partial(
    pl.kernel,
    out_type=jax.ShapeDtypeStruct((B, NSC), jnp.float32),
    mesh=plsc.VectorSubcoreMesh(core_axis_name="c", subcore_axis_name="s",
                                num_cores=NC, num_subcores=NS),
    scratch_types=[
        pltpu.VMEM((CA, B, 128), jnp.float32),
        pltpu.VMEM((CB, B, 128), jnp.float32),
        pltpu.VMEM((B * 128,), jnp.float32),
        pltpu.VMEM((2, B, 128), jnp.float32),
        pltpu.VMEM((B, 128), jnp.float32),
        pltpu.VMEM((B * ASTR * L,), jnp.float32),
        pltpu.SemaphoreType.DMA,
        pltpu.SemaphoreType.DMA,
        pltpu.SemaphoreType.DMA,
    ],
    compiler_params=pltpu.CompilerParams(use_tc_tiling_on_sc=True))(_sc_body)


NG = NTC // BN + 1
BB = 128 * (-(-N // (128 * NG)))


def _tc_main_body(ts_ref, prt_ref, lg_ref, bx_ref,
                  sco_ref, box_ref, lab_ref):
  x = lg_ref[...]
  a = 1.0 + jnp.exp(-prt_ref[...][:C])
  acc = jnp.min(a[:, :, None] * jnp.exp(-x) + a[:, :, None], axis=0)
  sco_ref[...] = 1.0 / acc
  xb = bx_ref[...]
  ts = ts_ref[...].astype(jnp.float32)
  hh = ts[0][:, None, None]
  ww = ts[1][:, None, None]
  coord = lax.broadcasted_iota(jnp.int32, (B, 4, BB), 1)
  half = jnp.where(coord >= 2, 0.5, -0.5)
  cxy = jnp.concatenate([xb[:, 0:2], xb[:, 0:2]], axis=1)
  wh = jnp.concatenate([xb[:, 2:4], xb[:, 2:4]], axis=1)
  scale = jnp.where(coord % 2 == 0, ww, hh)
  box_ref[...] = (cxy + half * wh) * scale
  lab_ref[...] = jnp.ones((B, BB), jnp.int32)


def _tc_main(lgt, prt, bxt, tst):
  return pl.pallas_call(
      _tc_main_body,
      grid=(NG,),
      in_specs=[
          pl.BlockSpec((2, B), lambda j: (0, 0)),
          pl.BlockSpec((128, B), lambda j: (0, 0)),
          pl.BlockSpec((C, B, BN), lambda j: (0, 0, j + NSC // BN)),
          pl.BlockSpec((B, 4, BB), lambda j: (0, 0, j)),
      ],
      out_specs=[
          pl.BlockSpec((B, BN), lambda j: (0, j)),
          pl.BlockSpec((B, 4, BB), lambda j: (0, 0, j)),
          pl.BlockSpec((B, BB), lambda j: (0, j)),
      ],
      out_shape=[
          jax.ShapeDtypeStruct((B, NTC), jnp.float32),
          jax.ShapeDtypeStruct((B, 4, N), jnp.float32),
          jax.ShapeDtypeStruct((B, N), jnp.int32),
      ],
  )(tst, prt, lgt, bxt)


def kernel(pred_logits, pred_boxes, presence_logit_dec,
           target_sizes_boxes, target_sizes_masks):
  del target_sizes_masks
  lgt = jnp.transpose(pred_logits, (2, 0, 1))
  bxt = jnp.transpose(pred_boxes, (0, 2, 1))
  tst = jnp.transpose(target_sizes_boxes, (1, 0))
  pr_pad = jnp.pad(presence_logit_dec, ((0, 0), (0, 128 - C)))
  prt = jnp.transpose(pr_pad, (1, 0))
  sc_part = _sc_scores(lgt, pr_pad)
  tc_part, boxes_t, labels = _tc_main(lgt, prt, bxt, tst)
  scores = jnp.concatenate([sc_part, tc_part], axis=1)
  boxes = jnp.transpose(boxes_t, (0, 2, 1))
  return scores, labels, boxes
